# baseline (speedup 1.0000x reference)
import numpy as np
import jax
import jax.numpy as jnp
from jax.experimental import pallas as pl

_LHAND = np.arange(468, 489)
_RHAND = np.arange(522, 543)
_REYE = np.array([33, 7, 163, 144, 145, 153, 154, 155, 133, 246, 161, 160, 159, 158, 157, 173])
_LEYE = np.array([263, 249, 390, 373, 374, 380, 381, 382, 362, 466, 388, 387, 386, 385, 384, 398])
_SLIP = np.array([78, 95, 88, 178, 87, 14, 317, 402, 318, 324, 308, 191, 80, 81, 82, 13, 312, 311, 310, 415])
_SPOSE = np.array([11, 13, 15, 12, 14, 16, 23, 24]) + 489
_TRIU = np.array([1, 2, 3, 4, 5, 6, 7, 8, 9, 10, 11, 12, 13, 14, 15, 16, 17, 18, 19, 20, 23, 24, 25, 26, 27, 28, 29, 30, 31, 32, 33, 34, 35, 36, 37, 38, 39, 40, 41, 45, 46, 47, 48, 49, 50, 51, 52, 53, 54, 55, 56, 57, 58, 59, 60, 61, 62, 67, 68, 69, 70, 71, 72, 73, 74, 75, 76, 77, 78, 79, 80, 81, 82, 83, 89, 90, 91, 92, 93, 94, 95, 96, 97, 98, 99, 100, 101, 102, 103, 104, 111, 112, 113, 114, 115, 116, 117, 118, 119, 120, 121, 122, 123, 124, 125, 133, 134, 135, 136, 137, 138, 139, 140, 141, 142, 143, 144, 145, 146, 155, 156, 157, 158, 159, 160, 161, 162, 163, 164, 165, 166, 167, 177, 178, 179, 180, 181, 182, 183, 184, 185, 186, 187, 188, 199, 200, 201, 202, 203, 204, 205, 206, 207, 208, 209, 221, 222, 223, 224, 225, 226, 227, 228, 229, 230, 243, 244, 245, 246, 247, 248, 249, 250, 251, 265, 266, 267, 268, 269, 270, 271, 272, 287, 288, 289, 290, 291, 292, 293, 309, 310, 311, 312, 313, 314, 331, 332, 333, 334, 335, 353, 354, 355, 356, 375, 376, 377, 397, 398, 419])

_NRAW = 256
_NFRAME = 100
_START = 78
_NIN = 1629
_NCAT = 204
_NPAIR = 210
_NOUT = 828
_NVALID = float(_NFRAME * 543 * 2)

_IDX102 = np.concatenate([_LHAND, _RHAND, _SPOSE, _LEYE, _REYE, _SLIP])
_PAIRS = [divmod(int(k), 21) for k in _TRIU]


def _build_consts():
    s = np.zeros((_NFRAME, _NRAW), np.float32)
    for t in range(_NFRAME):
        s[t, _START + t] = 1.0
    wcat = np.zeros((_NIN, _NCAT), np.float32)
    for j, idx in enumerate(_IDX102):
        wcat[3 * idx, 2 * j] = 1.0
        wcat[3 * idx + 1, 2 * j + 1] = 1.0
    wx = np.zeros((_NIN, 2 * _NPAIR), np.float32)
    wy = np.zeros((_NIN, 2 * _NPAIR), np.float32)
    for p, (i, j) in enumerate(_PAIRS):
        wx[3 * (522 + i), p] = 1.0
        wx[3 * (522 + j), p] = -1.0
        wy[3 * (522 + i) + 1, p] = 1.0
        wy[3 * (522 + j) + 1, p] = -1.0
        wx[3 * (468 + i), _NPAIR + p] = 1.0
        wx[3 * (468 + j), _NPAIR + p] = -1.0
        wy[3 * (468 + i) + 1, _NPAIR + p] = 1.0
        wy[3 * (468 + j) + 1, _NPAIR + p] = -1.0
    m = np.zeros((_NFRAME, _NFRAME), np.float32)
    for t in range(_NFRAME - 1):
        m[t, t] = 1.0
        m[t, t + 1] = -1.0
    to_bf = lambda a: a.astype(jnp.bfloat16)
    return to_bf(s), to_bf(wcat), to_bf(wx), to_bf(wy), to_bf(m)


_S, _WCAT, _WX, _WY, _M = _build_consts()


def _dot(a, b):
    return jax.lax.dot_general(
        a, b, (((1,), (0,)), ((), ())),
        preferred_element_type=jnp.float32)


def _body(x_ref, s_ref, wcat_ref, wx_ref, wy_ref, m_ref, o_ref):
    xb = x_ref[...].astype(jnp.bfloat16)
    xs = _dot(s_ref[...], xb)
    col = jax.lax.broadcasted_iota(jnp.int32, xs.shape, 1)
    valid = jax.lax.rem(col, 3) != 2
    s1 = jnp.sum(jnp.where(valid, xs, 0.0))
    s2 = jnp.sum(jnp.where(valid, xs * xs, 0.0))
    mean = s1 / _NVALID
    var = s2 / _NVALID - mean * mean
    rstd = jax.lax.rsqrt(var)
    xn = ((xs - mean) * rstd).astype(jnp.bfloat16)
    cat = _dot(xn, wcat_ref[...])
    ux = _dot(xn, wx_ref[...])
    uy = _dot(xn, wy_ref[...])
    dist = jnp.sqrt(ux * ux + uy * uy)
    dcat = _dot(m_ref[...], cat.astype(jnp.bfloat16))
    o_ref[...] = jnp.concatenate([cat, dcat, dist], axis=1)


@jax.jit
def kernel(xyz):
    xflat = xyz.reshape(_NRAW, _NIN)
    out = pl.pallas_call(
        _body,
        out_shape=jax.ShapeDtypeStruct((_NFRAME, _NOUT), jnp.float32),
    )(xflat, _S, _WCAT, _WX, _WY, _M)
    return out


# speedup vs baseline: 1.1657x; 1.1657x over previous
import numpy as np
import jax
import jax.numpy as jnp
from jax.experimental import pallas as pl

_LHAND = np.arange(468, 489)
_RHAND = np.arange(522, 543)
_REYE = np.array([33, 7, 163, 144, 145, 153, 154, 155, 133, 246, 161, 160, 159, 158, 157, 173])
_LEYE = np.array([263, 249, 390, 373, 374, 380, 381, 382, 362, 466, 388, 387, 386, 385, 384, 398])
_SLIP = np.array([78, 95, 88, 178, 87, 14, 317, 402, 318, 324, 308, 191, 80, 81, 82, 13, 312, 311, 310, 415])
_SPOSE = np.array([11, 13, 15, 12, 14, 16, 23, 24]) + 489
_TRIU = np.array([1, 2, 3, 4, 5, 6, 7, 8, 9, 10, 11, 12, 13, 14, 15, 16, 17, 18, 19, 20, 23, 24, 25, 26, 27, 28, 29, 30, 31, 32, 33, 34, 35, 36, 37, 38, 39, 40, 41, 45, 46, 47, 48, 49, 50, 51, 52, 53, 54, 55, 56, 57, 58, 59, 60, 61, 62, 67, 68, 69, 70, 71, 72, 73, 74, 75, 76, 77, 78, 79, 80, 81, 82, 83, 89, 90, 91, 92, 93, 94, 95, 96, 97, 98, 99, 100, 101, 102, 103, 104, 111, 112, 113, 114, 115, 116, 117, 118, 119, 120, 121, 122, 123, 124, 125, 133, 134, 135, 136, 137, 138, 139, 140, 141, 142, 143, 144, 145, 146, 155, 156, 157, 158, 159, 160, 161, 162, 163, 164, 165, 166, 167, 177, 178, 179, 180, 181, 182, 183, 184, 185, 186, 187, 188, 199, 200, 201, 202, 203, 204, 205, 206, 207, 208, 209, 221, 222, 223, 224, 225, 226, 227, 228, 229, 230, 243, 244, 245, 246, 247, 248, 249, 250, 251, 265, 266, 267, 268, 269, 270, 271, 272, 287, 288, 289, 290, 291, 292, 293, 309, 310, 311, 312, 313, 314, 331, 332, 333, 334, 335, 353, 354, 355, 356, 375, 376, 377, 397, 398, 419])

_NRAW = 256
_NFRAME = 100
_START = 78
_NIN = 1629
_NCAT = 204
_NPAIR = 210
_NOUT = 828
_NVALID = float(_NFRAME * 543 * 2)

_IDX102 = np.concatenate([_LHAND, _RHAND, _SPOSE, _LEYE, _REYE, _SLIP])
_PAIRS = [divmod(int(k), 21) for k in _TRIU]


def _build_consts():
    tgt = np.empty((_NCAT,), np.int32)
    for j, idx in enumerate(_IDX102):
        tgt[2 * j] = 3 * idx
        tgt[2 * j + 1] = 3 * idx + 1
    pxi = np.empty((2 * _NPAIR,), np.int32)
    pxj = np.empty((2 * _NPAIR,), np.int32)
    for p, (i, j) in enumerate(_PAIRS):
        pxi[p] = 42 + 2 * i
        pxj[p] = 42 + 2 * j
        pxi[_NPAIR + p] = 2 * i
        pxj[_NPAIR + p] = 2 * j
    return tgt, pxi, pxj


_TGT, _PXI, _PXJ = _build_consts()


def _dot(a, b):
    return jax.lax.dot_general(
        a, b, (((1,), (0,)), ((), ())),
        preferred_element_type=jnp.float32)


def _sel(rows, t):
    n = t.shape[-1]
    row = jax.lax.broadcasted_iota(jnp.int32, (rows, n), 0)
    return (row == t).astype(jnp.bfloat16)


def _body(x_ref, tgt_ref, pxi_ref, pxj_ref, o_ref):
    xb = x_ref[...].astype(jnp.bfloat16)
    srow = jax.lax.broadcasted_iota(jnp.int32, (_NFRAME, _NRAW), 0)
    scol = jax.lax.broadcasted_iota(jnp.int32, (_NFRAME, _NRAW), 1)
    s = (scol == srow + _START).astype(jnp.bfloat16)
    xs = _dot(s, xb)
    col = jax.lax.broadcasted_iota(jnp.int32, xs.shape, 1)
    valid = jax.lax.rem(col, 3) != 2
    s1 = jnp.sum(jnp.where(valid, xs, 0.0))
    s2 = jnp.sum(jnp.where(valid, xs * xs, 0.0))
    mean = s1 / _NVALID
    var = s2 / _NVALID - mean * mean
    rstd = jax.lax.rsqrt(var)
    xn = ((xs - mean) * rstd).astype(jnp.bfloat16)
    wcat = _sel(_NIN, tgt_ref[...])
    cat = _dot(xn, wcat)
    catb = cat.astype(jnp.bfloat16)
    prow = jax.lax.broadcasted_iota(jnp.int32, (_NCAT, 2 * _NPAIR), 0)
    ti = pxi_ref[...]
    tj = pxj_ref[...]
    px = ((prow == ti).astype(jnp.bfloat16) -
          (prow == tj).astype(jnp.bfloat16))
    py = ((prow == ti + 1).astype(jnp.bfloat16) -
          (prow == tj + 1).astype(jnp.bfloat16))
    ux = _dot(catb, px)
    uy = _dot(catb, py)
    dist = jnp.sqrt(ux * ux + uy * uy)
    mrow = jax.lax.broadcasted_iota(jnp.int32, (_NFRAME, _NFRAME), 0)
    mcol = jax.lax.broadcasted_iota(jnp.int32, (_NFRAME, _NFRAME), 1)
    m = jnp.where(mrow < _NFRAME - 1,
                  (mcol == mrow).astype(jnp.bfloat16) -
                  (mcol == mrow + 1).astype(jnp.bfloat16),
                  jnp.bfloat16(0))
    dcat = _dot(m, catb)
    o_ref[...] = jnp.concatenate([cat, dcat, dist], axis=1)


@jax.jit
def kernel(xyz):
    xflat = xyz.reshape(_NRAW, _NIN)
    out = pl.pallas_call(
        _body,
        out_shape=jax.ShapeDtypeStruct((_NFRAME, _NOUT), jnp.float32),
    )(xflat, _TGT.reshape(1, -1), _PXI.reshape(1, -1), _PXJ.reshape(1, -1))
    return out


# speedup vs baseline: 1.4854x; 1.2743x over previous
import numpy as np
import jax
import jax.numpy as jnp
from jax.experimental import pallas as pl

_LHAND = np.arange(468, 489)
_RHAND = np.arange(522, 543)
_REYE = np.array([33, 7, 163, 144, 145, 153, 154, 155, 133, 246, 161, 160, 159, 158, 157, 173])
_LEYE = np.array([263, 249, 390, 373, 374, 380, 381, 382, 362, 466, 388, 387, 386, 385, 384, 398])
_SLIP = np.array([78, 95, 88, 178, 87, 14, 317, 402, 318, 324, 308, 191, 80, 81, 82, 13, 312, 311, 310, 415])
_SPOSE = np.array([11, 13, 15, 12, 14, 16, 23, 24]) + 489
_TRIU = np.array([1, 2, 3, 4, 5, 6, 7, 8, 9, 10, 11, 12, 13, 14, 15, 16, 17, 18, 19, 20, 23, 24, 25, 26, 27, 28, 29, 30, 31, 32, 33, 34, 35, 36, 37, 38, 39, 40, 41, 45, 46, 47, 48, 49, 50, 51, 52, 53, 54, 55, 56, 57, 58, 59, 60, 61, 62, 67, 68, 69, 70, 71, 72, 73, 74, 75, 76, 77, 78, 79, 80, 81, 82, 83, 89, 90, 91, 92, 93, 94, 95, 96, 97, 98, 99, 100, 101, 102, 103, 104, 111, 112, 113, 114, 115, 116, 117, 118, 119, 120, 121, 122, 123, 124, 125, 133, 134, 135, 136, 137, 138, 139, 140, 141, 142, 143, 144, 145, 146, 155, 156, 157, 158, 159, 160, 161, 162, 163, 164, 165, 166, 167, 177, 178, 179, 180, 181, 182, 183, 184, 185, 186, 187, 188, 199, 200, 201, 202, 203, 204, 205, 206, 207, 208, 209, 221, 222, 223, 224, 225, 226, 227, 228, 229, 230, 243, 244, 245, 246, 247, 248, 249, 250, 251, 265, 266, 267, 268, 269, 270, 271, 272, 287, 288, 289, 290, 291, 292, 293, 309, 310, 311, 312, 313, 314, 331, 332, 333, 334, 335, 353, 354, 355, 356, 375, 376, 377, 397, 398, 419])

_NRAW = 256
_NFRAME = 100
_START = 78
_NIN = 1629
_NCAT = 204
_NPAIR = 210
_NOUT = 828
_NVALID = float(_NFRAME * 543 * 2)

_IDX102 = np.concatenate([_LHAND, _RHAND, _SPOSE, _LEYE, _REYE, _SLIP])
_PAIRS = [divmod(int(k), 21) for k in _TRIU]


def _build_consts():
    tgt = np.empty((_NCAT,), np.int32)
    for j, idx in enumerate(_IDX102):
        tgt[2 * j] = 3 * idx
        tgt[2 * j + 1] = 3 * idx + 1
    pxi = np.empty((2 * _NPAIR,), np.int32)
    pxj = np.empty((2 * _NPAIR,), np.int32)
    for p, (i, j) in enumerate(_PAIRS):
        pxi[p] = 42 + 2 * i
        pxj[p] = 42 + 2 * j
        pxi[_NPAIR + p] = 2 * i
        pxj[_NPAIR + p] = 2 * j
    return tgt, pxi, pxj


_TGT, _PXI, _PXJ = _build_consts()


def _dot(a, b):
    return jax.lax.dot_general(
        a, b, (((1,), (0,)), ((), ())),
        preferred_element_type=jnp.float32)


def _sel(rows, t):
    n = t.shape[-1]
    row = jax.lax.broadcasted_iota(jnp.int32, (rows, n), 0)
    return (row == t).astype(jnp.bfloat16)


def _body(x_ref, tgt_ref, pxi_ref, pxj_ref, o_ref):
    xs = x_ref[...]
    col = jax.lax.broadcasted_iota(jnp.int32, xs.shape, 1)
    valid = jax.lax.rem(col, 3) != 2
    s1 = jnp.sum(jnp.where(valid, xs, 0.0))
    s2 = jnp.sum(jnp.where(valid, xs * xs, 0.0))
    mean = s1 / _NVALID
    var = s2 / _NVALID - mean * mean
    rstd = jax.lax.rsqrt(var)
    xn = ((xs - mean) * rstd).astype(jnp.bfloat16)
    wcat = _sel(_NIN, tgt_ref[...])
    cat = _dot(xn, wcat)
    catb = cat.astype(jnp.bfloat16)
    prow = jax.lax.broadcasted_iota(jnp.int32, (_NCAT, 2 * _NPAIR), 0)
    ti = pxi_ref[...]
    tj = pxj_ref[...]
    px = ((prow == ti).astype(jnp.bfloat16) -
          (prow == tj).astype(jnp.bfloat16))
    py = ((prow == ti + 1).astype(jnp.bfloat16) -
          (prow == tj + 1).astype(jnp.bfloat16))
    ux = _dot(catb, px)
    uy = _dot(catb, py)
    dist = jnp.sqrt(ux * ux + uy * uy)
    mrow = jax.lax.broadcasted_iota(jnp.int32, (_NFRAME, _NFRAME), 0)
    mcol = jax.lax.broadcasted_iota(jnp.int32, (_NFRAME, _NFRAME), 1)
    m = jnp.where(mrow < _NFRAME - 1,
                  (mcol == mrow).astype(jnp.bfloat16) -
                  (mcol == mrow + 1).astype(jnp.bfloat16),
                  jnp.bfloat16(0))
    dcat = _dot(m, catb)
    o_ref[...] = jnp.concatenate([cat, dcat, dist], axis=1)


@jax.jit
def kernel(xyz):
    xflat = xyz[_START:_START + _NFRAME].reshape(_NFRAME, _NIN)
    out = pl.pallas_call(
        _body,
        out_shape=jax.ShapeDtypeStruct((_NFRAME, _NOUT), jnp.float32),
    )(xflat, _TGT.reshape(1, -1), _PXI.reshape(1, -1), _PXJ.reshape(1, -1))
    return out


# speedup vs baseline: 1.6825x; 1.1327x over previous
import numpy as np
import jax
import jax.numpy as jnp
from jax.experimental import pallas as pl

_LHAND = np.arange(468, 489)
_RHAND = np.arange(522, 543)
_REYE = np.array([33, 7, 163, 144, 145, 153, 154, 155, 133, 246, 161, 160, 159, 158, 157, 173])
_LEYE = np.array([263, 249, 390, 373, 374, 380, 381, 382, 362, 466, 388, 387, 386, 385, 384, 398])
_SLIP = np.array([78, 95, 88, 178, 87, 14, 317, 402, 318, 324, 308, 191, 80, 81, 82, 13, 312, 311, 310, 415])
_SPOSE = np.array([11, 13, 15, 12, 14, 16, 23, 24]) + 489
_TRIU = np.array([1, 2, 3, 4, 5, 6, 7, 8, 9, 10, 11, 12, 13, 14, 15, 16, 17, 18, 19, 20, 23, 24, 25, 26, 27, 28, 29, 30, 31, 32, 33, 34, 35, 36, 37, 38, 39, 40, 41, 45, 46, 47, 48, 49, 50, 51, 52, 53, 54, 55, 56, 57, 58, 59, 60, 61, 62, 67, 68, 69, 70, 71, 72, 73, 74, 75, 76, 77, 78, 79, 80, 81, 82, 83, 89, 90, 91, 92, 93, 94, 95, 96, 97, 98, 99, 100, 101, 102, 103, 104, 111, 112, 113, 114, 115, 116, 117, 118, 119, 120, 121, 122, 123, 124, 125, 133, 134, 135, 136, 137, 138, 139, 140, 141, 142, 143, 144, 145, 146, 155, 156, 157, 158, 159, 160, 161, 162, 163, 164, 165, 166, 167, 177, 178, 179, 180, 181, 182, 183, 184, 185, 186, 187, 188, 199, 200, 201, 202, 203, 204, 205, 206, 207, 208, 209, 221, 222, 223, 224, 225, 226, 227, 228, 229, 230, 243, 244, 245, 246, 247, 248, 249, 250, 251, 265, 266, 267, 268, 269, 270, 271, 272, 287, 288, 289, 290, 291, 292, 293, 309, 310, 311, 312, 313, 314, 331, 332, 333, 334, 335, 353, 354, 355, 356, 375, 376, 377, 397, 398, 419])

_NRAW = 256
_NFRAME = 100
_START = 78
_NIN = 1086
_NCAT = 204
_NPAIR = 210
_NOUT = 828
_NVALID = float(_NFRAME * 543 * 2)

_IDX102 = np.concatenate([_LHAND, _RHAND, _SPOSE, _LEYE, _REYE, _SLIP])
_PAIRS = [divmod(int(k), 21) for k in _TRIU]


def _build_consts():
    tgt = np.empty((_NCAT,), np.int32)
    for j, idx in enumerate(_IDX102):
        tgt[2 * j] = 2 * idx
        tgt[2 * j + 1] = 2 * idx + 1
    pxi = np.empty((2 * _NPAIR,), np.int32)
    pxj = np.empty((2 * _NPAIR,), np.int32)
    for p, (i, j) in enumerate(_PAIRS):
        pxi[p] = 42 + 2 * i
        pxj[p] = 42 + 2 * j
        pxi[_NPAIR + p] = 2 * i
        pxj[_NPAIR + p] = 2 * j
    return tgt, pxi, pxj


_TGT, _PXI, _PXJ = _build_consts()


def _dot(a, b):
    return jax.lax.dot_general(
        a, b, (((1,), (0,)), ((), ())),
        preferred_element_type=jnp.float32)


def _sel(rows, t):
    n = t.shape[-1]
    row = jax.lax.broadcasted_iota(jnp.int32, (rows, n), 0)
    return (row == t).astype(jnp.bfloat16)


def _body(x_ref, tgt_ref, pxi_ref, pxj_ref, o_ref):
    xs = x_ref[...]
    s1 = jnp.sum(xs)
    s2 = jnp.sum(xs * xs)
    mean = s1 / _NVALID
    var = s2 / _NVALID - mean * mean
    rstd = jax.lax.rsqrt(var)
    xn = ((xs - mean) * rstd).astype(jnp.bfloat16)
    wcat = _sel(_NIN, tgt_ref[...])
    cat = _dot(xn, wcat)
    catb = cat.astype(jnp.bfloat16)
    prow = jax.lax.broadcasted_iota(jnp.int32, (_NCAT, 2 * _NPAIR), 0)
    ti = pxi_ref[...]
    tj = pxj_ref[...]
    px = ((prow == ti).astype(jnp.bfloat16) -
          (prow == tj).astype(jnp.bfloat16))
    py = ((prow == ti + 1).astype(jnp.bfloat16) -
          (prow == tj + 1).astype(jnp.bfloat16))
    ux = _dot(catb, px)
    uy = _dot(catb, py)
    dist = jnp.sqrt(ux * ux + uy * uy)
    mrow = jax.lax.broadcasted_iota(jnp.int32, (_NFRAME, _NFRAME), 0)
    mcol = jax.lax.broadcasted_iota(jnp.int32, (_NFRAME, _NFRAME), 1)
    m = jnp.where(mrow < _NFRAME - 1,
                  (mcol == mrow).astype(jnp.bfloat16) -
                  (mcol == mrow + 1).astype(jnp.bfloat16),
                  jnp.bfloat16(0))
    dcat = _dot(m, catb)
    o_ref[...] = jnp.concatenate([cat, dcat, dist], axis=1)


@jax.jit
def kernel(xyz):
    xflat = xyz[_START:_START + _NFRAME, :, :2].reshape(_NFRAME, _NIN)
    out = pl.pallas_call(
        _body,
        out_shape=jax.ShapeDtypeStruct((_NFRAME, _NOUT), jnp.float32),
    )(xflat, _TGT.reshape(1, -1), _PXI.reshape(1, -1), _PXJ.reshape(1, -1))
    return out
